# baseline (speedup 1.0000x reference)
import functools

import jax
import jax.numpy as jnp
from jax import lax
from jax.experimental import pallas as pl
from jax.experimental.pallas import tpu as pltpu
from jax.experimental.pallas import tpu_sc as plsc

DIM = 128
K = 100000
BATCH = 16384

NC = 2
NS = 16
NCW = 16
RPW = DIM // NCW
CW = 5248
NCHUNK = 15
LASTW = K - BATCH - NCHUNK * CW

TBLK = 2048
NTBLK = BATCH // TBLK


def _sc_copy_body(q_hbm, o_hbm, buf, lastbuf, isem, osem, lsem):
    wid = lax.axis_index("s") * NC + lax.axis_index("c")

    @pl.when(wid < NCW)
    def _():
        r0 = pl.multiple_of(wid * RPW, RPW)

        def din(j):
            co = BATCH + j * CW
            return pltpu.make_async_copy(
                q_hbm.at[pl.ds(r0, RPW), pl.ds(co, CW)], buf.at[j % 2],
                isem.at[j % 2])

        def dout(j):
            co = BATCH + j * CW
            return pltpu.make_async_copy(
                buf.at[j % 2], o_hbm.at[pl.ds(r0, RPW), pl.ds(co, CW)],
                osem.at[j % 2])

        def lin():
            return pltpu.make_async_copy(
                q_hbm.at[pl.ds(r0, RPW), pl.ds(K - LASTW, LASTW)],
                lastbuf, lsem.at[0])

        def lout():
            return pltpu.make_async_copy(
                lastbuf, o_hbm.at[pl.ds(r0, RPW), pl.ds(K - LASTW, LASTW)],
                lsem.at[1])

        lin().start()
        din(0).start()
        for j in range(NCHUNK):
            if j + 1 < NCHUNK:
                if j >= 1:
                    dout(j - 1).wait()
                din(j + 1).start()
            din(j).wait()
            dout(j).start()
            if j == 1:
                lin().wait()
                lout().start()
        dout(NCHUNK - 2).wait()
        dout(NCHUNK - 1).wait()
        lout().wait()


_sc_copy = functools.partial(
    pl.kernel,
    out_type=jax.ShapeDtypeStruct((DIM, K), jnp.float32),
    mesh=plsc.VectorSubcoreMesh(core_axis_name="c", subcore_axis_name="s"),
    scratch_types=[
        pltpu.VMEM((2, RPW, CW), jnp.float32),
        pltpu.VMEM((RPW, LASTW), jnp.float32),
        pltpu.SemaphoreType.DMA((2,)),
        pltpu.SemaphoreType.DMA((2,)),
        pltpu.SemaphoreType.DMA((2,)),
    ],
)(_sc_copy_body)


def _xpose_body(k_ref, _, o_ref):
    o_ref[...] = k_ref[...].T


def kernel(keys, queue):
    tail = _sc_copy(queue)

    new_queue = pl.pallas_call(
        _xpose_body,
        grid=(NTBLK,),
        in_specs=[
            pl.BlockSpec((TBLK, DIM), lambda i: (i, 0)),
            pl.BlockSpec(memory_space=pl.ANY),
        ],
        out_specs=pl.BlockSpec((DIM, TBLK), lambda i: (0, i)),
        out_shape=jax.ShapeDtypeStruct((DIM, K), jnp.float32),
        input_output_aliases={1: 0},
    )(keys, tail)

    new_ptr = jnp.array([BATCH % K], dtype=jnp.int32)
    return new_queue, new_ptr
